# Initial kernel scaffold; baseline (speedup 1.0000x reference)
import jax
import jax.numpy as jnp
from jax.experimental import pallas as pl


def kernel(X, C):
    raise NotImplementedError("write your pallas kernel here")


# speedup vs baseline: 15.1976x; 15.1976x over previous
import dataclasses
import functools

import jax
import jax.numpy as jnp
from jax import lax
from jax.experimental import pallas as pl
from jax.experimental.pallas import tpu as pltpu
from jax.experimental.pallas import tpu_sc as plsc

_NUM_NEIGHBORS = 30
_SMOOTH_STEPS = 5
_NORM_EPS = 1e-5
_DET_SEED = 10
_BIG = 1e9

_RB = 256
_ROWS_PER_SUBCORE = 256
_LANES = 16

_DOT_MODE = "mxu"
_X2_MODE = "fold"
_K_MODE = "xla"


def _sc_compiler_params():
    cp = pltpu.CompilerParams()
    if "needs_layout_passes" in pltpu.CompilerParams.__dataclass_fields__:
        cp = dataclasses.replace(cp, needs_layout_passes=False)
    return cp


def _x2_sum(p0, p1, p2):
    if _X2_MODE == "fold":
        return (p0 + p2) + p1
    return (p0 + p1) + p2


def _tree_sum_k(ts):
    n = len(ts)
    if _K_MODE == "xla":
        u = []
        for s in range(8):
            acc = ts[s]
            j = s + 8
            while j < n:
                acc = acc + ts[j]
                j += 8
            u.append(acc)
        w = [u[s] + u[s + 4] for s in range(4)]
        return (w[0] + w[2]) + (w[1] + w[3])
    if _K_MODE == "seq":
        acc = ts[0]
        for t in ts[1:]:
            acc = acc + t
        return acc
    if _K_MODE == "fold":
        cur = list(ts) + [None] * (32 - n)
        width = 32
        while width > 1:
            half = width // 2
            nxt = []
            for i in range(half):
                a, b = cur[i], cur[i + half]
                if a is None:
                    nxt.append(b)
                elif b is None:
                    nxt.append(a)
                else:
                    nxt.append(a + b)
            cur = nxt
            width = half
        return cur[0]
    cur = list(ts)
    while len(cur) > 1:
        nxt = []
        for i in range(0, len(cur) - 1, 2):
            nxt.append(cur[i] + cur[i + 1])
        if len(cur) % 2:
            nxt.append(cur[-1])
        cur = nxt
    return cur[0]


def _topk_body(xr_ref, xt_ref, vr_ref, vc_ref, ei_ref, mk_ref):
    xr = xr_ref[0]
    xt = xt_ref[0]
    vr = vr_ref[0]
    vc = vc_ref[0]
    n = xt.shape[1]

    a0, a1, a2 = xr[:, 0:1], xr[:, 1:2], xr[:, 2:3]
    x2r = _x2_sum(a0 * a0, a1 * a1, a2 * a2)
    b0, b1, b2 = xt[0:1, :], xt[1:2, :], xt[2:3, :]
    x2c = _x2_sum(b0 * b0, b1 * b1, b2 * b2)

    if _DOT_MODE == "mxu":
        dot = lax.dot_general(
            xr, xt, (((1,), (0,)), ((), ())),
            precision=lax.Precision.DEFAULT,
            preferred_element_type=jnp.float32)
    else:
        dot = (a0 * b0 + a1 * b1) + a2 * b2

    d2 = (x2r + x2c) - 2.0 * dot
    m2 = vr * vc
    work = jnp.where(m2 > 0, d2, jnp.float32(_BIG))

    iota_i = lax.broadcasted_iota(jnp.int32, (1, n), 1)
    inf = jnp.float32(jnp.inf)
    big_i = jnp.int32(2**30)
    for k in range(_NUM_NEIGHBORS):
        mn = jnp.min(work, axis=1, keepdims=True)
        cand = jnp.where(work == mn, iota_i, big_i)
        idx = jnp.min(cand, axis=1, keepdims=True)
        ei_ref[0, k, :] = idx[:, 0]
        mk_ref[0, k, :] = jnp.where(mn[:, 0] < _BIG, 1.0, 0.0).astype(jnp.float32)
        work = jnp.where(iota_i == idx, inf, work)


_KPAD = 128


def _topk_edges(xca, xca_t, valid):
    b, n, _ = xca.shape
    grid = (b, n // _RB)
    return pl.pallas_call(
        _topk_body,
        grid=grid,
        in_specs=[
            pl.BlockSpec((1, _RB, _KPAD), lambda i, j: (i, j, 0)),
            pl.BlockSpec((1, _KPAD, n), lambda i, j: (i, 0, 0)),
            pl.BlockSpec((1, _RB, 1), lambda i, j: (i, j, 0)),
            pl.BlockSpec((1, 1, n), lambda i, j: (i, 0, 0)),
        ],
        out_specs=[
            pl.BlockSpec((1, _NUM_NEIGHBORS, _RB), lambda i, j: (i, 0, j)),
            pl.BlockSpec((1, _NUM_NEIGHBORS, _RB), lambda i, j: (i, 0, j)),
        ],
        out_shape=[
            jax.ShapeDtypeStruct((b, _NUM_NEIGHBORS, n), jnp.int32),
            jax.ShapeDtypeStruct((b, _NUM_NEIGHBORS, n), jnp.float32),
        ],
    )(_pad_k(xca), jnp.transpose(_pad_k(xca), (0, 2, 1)),
      valid[:, :, None], valid[:, None, :])


def _pad_k(x):
    b, n, d = x.shape
    return jnp.concatenate(
        [x, jnp.zeros((b, n, _KPAD - d), x.dtype)], axis=-1)


def _smooth_sc(z0, edge_t, mask_t):
    b, n = z0.shape
    rps = _ROWS_PER_SUBCORE
    k = _NUM_NEIGHBORS
    mesh = plsc.VectorSubcoreMesh(core_axis_name="c", subcore_axis_name="s")

    @functools.partial(
        pl.kernel,
        out_type=jax.ShapeDtypeStruct((b, n), jnp.float32),
        mesh=mesh,
        scratch_types=[
            pltpu.VMEM((n,), jnp.float32),
            pltpu.VMEM((k, rps), jnp.int32),
            pltpu.VMEM((k, rps), jnp.float32),
            pltpu.VMEM((rps,), jnp.float32),
            pltpu.VMEM_SHARED((n,), jnp.float32),
        ],
        compiler_params=_sc_compiler_params(),
    )
    def smooth(z0_hbm, edge_hbm, mask_hbm, zout_hbm,
               z_loc, edge_loc, mask_loc, z_new, z_shared):
        c = lax.axis_index("c")
        s = lax.axis_index("s")
        base = s * rps
        pltpu.sync_copy(z0_hbm.at[c], z_loc)
        pltpu.sync_copy(edge_hbm.at[c, :, pl.ds(base, rps)], edge_loc)
        pltpu.sync_copy(mask_hbm.at[c, :, pl.ds(base, rps)], mask_loc)

        for _step in range(_SMOOTH_STEPS):
            @pl.loop(0, rps // _LANES)
            def _(g):
                off = g * _LANES
                terms = []
                msums = []
                for kk in range(k):
                    idx = edge_loc[kk, pl.ds(off, _LANES)]
                    mk = mask_loc[kk, pl.ds(off, _LANES)]
                    zg = plsc.load_gather(z_loc, [idx])
                    terms.append(zg * mk)
                    msums.append(mk)
                ssum = _tree_sum_k(terms)
                msum = _tree_sum_k(msums)
                z_new[pl.ds(off, _LANES)] = ssum / (msum + jnp.float32(_NORM_EPS))

            pltpu.sync_copy(z_new, z_shared.at[pl.ds(base, rps)])
            plsc.subcore_barrier()
            pltpu.sync_copy(z_shared, z_loc)
            plsc.subcore_barrier()

        @pl.when(s == 0)
        def _():
            pltpu.sync_copy(z_loc, zout_hbm.at[c])

    return smooth(z0, edge_t, mask_t)


def _rank_body(zr_ref, zc_ref, out_ref):
    j = pl.program_id(1)
    zr = zr_ref[0]
    zc = zc_ref[0]
    n = zc.shape[1]
    iota_j = lax.broadcasted_iota(jnp.int32, (1, n), 1)
    row_idx = j * _RB + lax.broadcasted_iota(jnp.int32, (_RB, 1), 0)
    lt = zc < zr
    eq_before = (zc == zr) & (iota_j < row_idx)
    cnt = jnp.sum(jnp.where(lt | eq_before, 1, 0).astype(jnp.int32),
                  axis=1)
    out_ref[0, 0, 0, :] = cnt


def _ranks(z):
    b, n = z.shape
    nblk = n // _RB
    out = pl.pallas_call(
        _rank_body,
        grid=(b, nblk),
        in_specs=[
            pl.BlockSpec((1, _RB, 1), lambda i, j: (i, j, 0)),
            pl.BlockSpec((1, 1, n), lambda i, j: (i, 0, 0)),
        ],
        out_specs=pl.BlockSpec((1, 1, 1, _RB), lambda i, j: (i, j, 0, 0)),
        out_shape=jax.ShapeDtypeStruct((b, nblk, 1, _RB), jnp.int32),
    )(z[:, :, None], z[:, None, :])
    return out.reshape(b, n)


def _invert_perm_sc(rank):
    b, n = rank.shape
    mesh = plsc.VectorSubcoreMesh(core_axis_name="c", subcore_axis_name="s")

    @functools.partial(
        pl.kernel,
        out_type=jax.ShapeDtypeStruct((b, n), jnp.int32),
        mesh=mesh,
        scratch_types=[
            pltpu.VMEM((n,), jnp.int32),
            pltpu.VMEM((n,), jnp.int32),
        ],
        compiler_params=_sc_compiler_params(),
    )
    def invert(rank_hbm, out_hbm, idx_loc, out_loc):
        c = lax.axis_index("c")
        s = lax.axis_index("s")

        @pl.when(s == 0)
        def _():
            pltpu.sync_copy(rank_hbm.at[c], idx_loc)

            @pl.loop(0, n // _LANES)
            def _(i):
                off = i * _LANES
                idx = idx_loc[pl.ds(off, _LANES)]
                vals = lax.iota(jnp.int32, _LANES) + off
                plsc.store_scatter(out_loc, [idx], vals)

            pltpu.sync_copy(out_loc, out_hbm.at[c])

    return invert(rank)


def kernel(X, C):
    b, n = C.shape
    xca = X[:, :, 1, :]
    xca_t = jnp.transpose(xca, (0, 2, 1))
    valid = (C > 0).astype(jnp.float32)

    z = jax.random.uniform(jax.random.key(_DET_SEED), (1, n),
                           dtype=jnp.float32)
    z = jnp.broadcast_to(z, (b, n))

    edge_t, mask_t = _topk_edges(xca, xca_t, valid)
    z = _smooth_sc(z, edge_t, mask_t)
    rank = _ranks(z)
    return _invert_perm_sc(rank)


# speedup vs baseline: 15.2010x; 1.0002x over previous
import dataclasses
import functools

import jax
import jax.numpy as jnp
from jax import lax
from jax.experimental import pallas as pl
from jax.experimental.pallas import tpu as pltpu
from jax.experimental.pallas import tpu_sc as plsc

_NUM_NEIGHBORS = 30
_SMOOTH_STEPS = 5
_NORM_EPS = 1e-5
_DET_SEED = 10
_BIG = 1e9

_RB = 256
_ROWS_PER_SUBCORE = 256
_LANES = 16

_DOT_MODE = "mxu"
_X2_MODE = "fold"
_K_MODE = "xla"


def _sc_compiler_params():
    cp = pltpu.CompilerParams()
    if "needs_layout_passes" in pltpu.CompilerParams.__dataclass_fields__:
        cp = dataclasses.replace(cp, needs_layout_passes=False)
    return cp


def _x2_sum(p0, p1, p2):
    if _X2_MODE == "fold":
        return (p0 + p2) + p1
    return (p0 + p1) + p2


def _tree_sum_k(ts):
    n = len(ts)
    if _K_MODE == "xla":
        u = []
        for s in range(8):
            acc = ts[s]
            j = s + 8
            while j < n:
                acc = acc + ts[j]
                j += 8
            u.append(acc)
        w = [u[s] + u[s + 4] for s in range(4)]
        return (w[0] + w[2]) + (w[1] + w[3])
    if _K_MODE == "seq":
        acc = ts[0]
        for t in ts[1:]:
            acc = acc + t
        return acc
    if _K_MODE == "fold":
        cur = list(ts) + [None] * (32 - n)
        width = 32
        while width > 1:
            half = width // 2
            nxt = []
            for i in range(half):
                a, b = cur[i], cur[i + half]
                if a is None:
                    nxt.append(b)
                elif b is None:
                    nxt.append(a)
                else:
                    nxt.append(a + b)
            cur = nxt
            width = half
        return cur[0]
    cur = list(ts)
    while len(cur) > 1:
        nxt = []
        for i in range(0, len(cur) - 1, 2):
            nxt.append(cur[i] + cur[i + 1])
        if len(cur) % 2:
            nxt.append(cur[-1])
        cur = nxt
    return cur[0]


def _topk_body(xr_ref, xt_ref, vr_ref, vc_ref, ei_ref, mk_ref):
    xr = xr_ref[0]
    xt = xt_ref[0]
    vr = vr_ref[0]
    vc = vc_ref[0]
    n = xt.shape[1]

    a0, a1, a2 = xr[:, 0:1], xr[:, 1:2], xr[:, 2:3]
    x2r = _x2_sum(a0 * a0, a1 * a1, a2 * a2)
    b0, b1, b2 = xt[0:1, :], xt[1:2, :], xt[2:3, :]
    x2c = _x2_sum(b0 * b0, b1 * b1, b2 * b2)

    if _DOT_MODE == "mxu":
        dot = lax.dot_general(
            xr, xt, (((1,), (0,)), ((), ())),
            precision=lax.Precision.DEFAULT,
            preferred_element_type=jnp.float32)
    else:
        dot = (a0 * b0 + a1 * b1) + a2 * b2

    d2 = (x2r + x2c) - 2.0 * dot
    m2 = vr * vc
    work = jnp.where(m2 > 0, d2, jnp.float32(_BIG))

    iota_i = lax.broadcasted_iota(jnp.int32, (1, n), 1)
    inf = jnp.float32(jnp.inf)
    big_i = jnp.int32(2**30)
    for k in range(_NUM_NEIGHBORS):
        mn = jnp.min(work, axis=1, keepdims=True)
        cand = jnp.where(work == mn, iota_i, big_i)
        idx = jnp.min(cand, axis=1, keepdims=True)
        ei_ref[0, k, :] = idx[:, 0]
        mk_ref[0, k, :] = jnp.where(mn[:, 0] < _BIG, 1.0, 0.0).astype(jnp.float32)
        work = jnp.where(iota_i == idx, inf, work)


_KPAD = 128


def _topk_edges(xca, xca_t, valid):
    b, n, _ = xca.shape
    grid = (b, n // _RB)
    return pl.pallas_call(
        _topk_body,
        grid=grid,
        in_specs=[
            pl.BlockSpec((1, _RB, _KPAD), lambda i, j: (i, j, 0)),
            pl.BlockSpec((1, _KPAD, n), lambda i, j: (i, 0, 0)),
            pl.BlockSpec((1, _RB, 1), lambda i, j: (i, j, 0)),
            pl.BlockSpec((1, 1, n), lambda i, j: (i, 0, 0)),
        ],
        out_specs=[
            pl.BlockSpec((1, _NUM_NEIGHBORS, _RB), lambda i, j: (i, 0, j)),
            pl.BlockSpec((1, _NUM_NEIGHBORS, _RB), lambda i, j: (i, 0, j)),
        ],
        out_shape=[
            jax.ShapeDtypeStruct((b, _NUM_NEIGHBORS, n), jnp.int32),
            jax.ShapeDtypeStruct((b, _NUM_NEIGHBORS, n), jnp.float32),
        ],
        compiler_params=pltpu.CompilerParams(
            dimension_semantics=("parallel", "parallel")),
    )(_pad_k(xca), jnp.transpose(_pad_k(xca), (0, 2, 1)),
      valid[:, :, None], valid[:, None, :])


def _pad_k(x):
    b, n, d = x.shape
    return jnp.concatenate(
        [x, jnp.zeros((b, n, _KPAD - d), x.dtype)], axis=-1)


def _smooth_sc(z0, edge_t, mask_t):
    b, n = z0.shape
    rps = _ROWS_PER_SUBCORE
    k = _NUM_NEIGHBORS
    mesh = plsc.VectorSubcoreMesh(core_axis_name="c", subcore_axis_name="s")

    @functools.partial(
        pl.kernel,
        out_type=jax.ShapeDtypeStruct((b, n), jnp.float32),
        mesh=mesh,
        scratch_types=[
            pltpu.VMEM((n,), jnp.float32),
            pltpu.VMEM((k, rps), jnp.int32),
            pltpu.VMEM((k, rps), jnp.float32),
            pltpu.VMEM((rps,), jnp.float32),
            pltpu.VMEM_SHARED((n,), jnp.float32),
        ],
        compiler_params=_sc_compiler_params(),
    )
    def smooth(z0_hbm, edge_hbm, mask_hbm, zout_hbm,
               z_loc, edge_loc, mask_loc, z_new, z_shared):
        c = lax.axis_index("c")
        s = lax.axis_index("s")
        base = s * rps
        pltpu.sync_copy(z0_hbm.at[c], z_loc)
        pltpu.sync_copy(edge_hbm.at[c, :, pl.ds(base, rps)], edge_loc)
        pltpu.sync_copy(mask_hbm.at[c, :, pl.ds(base, rps)], mask_loc)

        for _step in range(_SMOOTH_STEPS):
            @pl.loop(0, rps // _LANES)
            def _(g):
                off = g * _LANES
                terms = []
                msums = []
                for kk in range(k):
                    idx = edge_loc[kk, pl.ds(off, _LANES)]
                    mk = mask_loc[kk, pl.ds(off, _LANES)]
                    zg = plsc.load_gather(z_loc, [idx])
                    terms.append(zg * mk)
                    msums.append(mk)
                ssum = _tree_sum_k(terms)
                msum = _tree_sum_k(msums)
                z_new[pl.ds(off, _LANES)] = ssum / (msum + jnp.float32(_NORM_EPS))

            pltpu.sync_copy(z_new, z_shared.at[pl.ds(base, rps)])
            plsc.subcore_barrier()
            pltpu.sync_copy(z_shared, z_loc)
            plsc.subcore_barrier()

        @pl.when(s == 0)
        def _():
            pltpu.sync_copy(z_loc, zout_hbm.at[c])

    return smooth(z0, edge_t, mask_t)


def _rank_body(zr_ref, zc_ref, out_ref):
    j = pl.program_id(1)
    zr = zr_ref[0]
    zc = zc_ref[0]
    n = zc.shape[1]
    iota_j = lax.broadcasted_iota(jnp.int32, (1, n), 1)
    row_idx = j * _RB + lax.broadcasted_iota(jnp.int32, (_RB, 1), 0)
    lt = zc < zr
    eq_before = (zc == zr) & (iota_j < row_idx)
    cnt = jnp.sum(jnp.where(lt | eq_before, 1, 0).astype(jnp.int32),
                  axis=1)
    out_ref[0, 0, 0, :] = cnt


def _ranks(z):
    b, n = z.shape
    nblk = n // _RB
    out = pl.pallas_call(
        _rank_body,
        grid=(b, nblk),
        in_specs=[
            pl.BlockSpec((1, _RB, 1), lambda i, j: (i, j, 0)),
            pl.BlockSpec((1, 1, n), lambda i, j: (i, 0, 0)),
        ],
        out_specs=pl.BlockSpec((1, 1, 1, _RB), lambda i, j: (i, j, 0, 0)),
        out_shape=jax.ShapeDtypeStruct((b, nblk, 1, _RB), jnp.int32),
        compiler_params=pltpu.CompilerParams(
            dimension_semantics=("parallel", "parallel")),
    )(z[:, :, None], z[:, None, :])
    return out.reshape(b, n)


def _invert_perm_sc(rank):
    b, n = rank.shape
    mesh = plsc.VectorSubcoreMesh(core_axis_name="c", subcore_axis_name="s")

    @functools.partial(
        pl.kernel,
        out_type=jax.ShapeDtypeStruct((b, n), jnp.int32),
        mesh=mesh,
        scratch_types=[
            pltpu.VMEM((n,), jnp.int32),
            pltpu.VMEM((n,), jnp.int32),
        ],
        compiler_params=_sc_compiler_params(),
    )
    def invert(rank_hbm, out_hbm, idx_loc, out_loc):
        c = lax.axis_index("c")
        s = lax.axis_index("s")

        @pl.when(s == 0)
        def _():
            pltpu.sync_copy(rank_hbm.at[c], idx_loc)

            @pl.loop(0, n // _LANES)
            def _(i):
                off = i * _LANES
                idx = idx_loc[pl.ds(off, _LANES)]
                vals = lax.iota(jnp.int32, _LANES) + off
                plsc.store_scatter(out_loc, [idx], vals)

            pltpu.sync_copy(out_loc, out_hbm.at[c])

    return invert(rank)


def kernel(X, C):
    b, n = C.shape
    xca = X[:, :, 1, :]
    xca_t = jnp.transpose(xca, (0, 2, 1))
    valid = (C > 0).astype(jnp.float32)

    z = jax.random.uniform(jax.random.key(_DET_SEED), (1, n),
                           dtype=jnp.float32)
    z = jnp.broadcast_to(z, (b, n))

    edge_t, mask_t = _topk_edges(xca, xca_t, valid)
    z = _smooth_sc(z, edge_t, mask_t)
    rank = _ranks(z)
    return _invert_perm_sc(rank)


# speedup vs baseline: 16.8370x; 1.1076x over previous
import dataclasses
import functools

import jax
import jax.numpy as jnp
from jax import lax
from jax.experimental import pallas as pl
from jax.experimental.pallas import tpu as pltpu
from jax.experimental.pallas import tpu_sc as plsc

_NUM_NEIGHBORS = 30
_SMOOTH_STEPS = 5
_NORM_EPS = 1e-5
_DET_SEED = 10
_BIG = 1e9

_RB = 512
_ROWS_PER_SUBCORE = 256
_LANES = 16

_DOT_MODE = "mxu"
_X2_MODE = "fold"
_K_MODE = "xla"


def _sc_compiler_params():
    cp = pltpu.CompilerParams()
    if "needs_layout_passes" in pltpu.CompilerParams.__dataclass_fields__:
        cp = dataclasses.replace(cp, needs_layout_passes=False)
    return cp


def _x2_sum(p0, p1, p2):
    if _X2_MODE == "fold":
        return (p0 + p2) + p1
    return (p0 + p1) + p2


def _tree_sum_k(ts):
    n = len(ts)
    if _K_MODE == "xla":
        u = []
        for s in range(8):
            acc = ts[s]
            j = s + 8
            while j < n:
                acc = acc + ts[j]
                j += 8
            u.append(acc)
        w = [u[s] + u[s + 4] for s in range(4)]
        return (w[0] + w[2]) + (w[1] + w[3])
    if _K_MODE == "seq":
        acc = ts[0]
        for t in ts[1:]:
            acc = acc + t
        return acc
    if _K_MODE == "fold":
        cur = list(ts) + [None] * (32 - n)
        width = 32
        while width > 1:
            half = width // 2
            nxt = []
            for i in range(half):
                a, b = cur[i], cur[i + half]
                if a is None:
                    nxt.append(b)
                elif b is None:
                    nxt.append(a)
                else:
                    nxt.append(a + b)
            cur = nxt
            width = half
        return cur[0]
    cur = list(ts)
    while len(cur) > 1:
        nxt = []
        for i in range(0, len(cur) - 1, 2):
            nxt.append(cur[i] + cur[i + 1])
        if len(cur) % 2:
            nxt.append(cur[-1])
        cur = nxt
    return cur[0]


def _topk_body(xr_ref, xt_ref, vr_ref, vc_ref, ei_ref, mk_ref):
    xr = xr_ref[0]
    xt = xt_ref[0]
    vr = vr_ref[0]
    vc = vc_ref[0]
    n = xt.shape[1]

    a0, a1, a2 = xr[:, 0:1], xr[:, 1:2], xr[:, 2:3]
    x2r = _x2_sum(a0 * a0, a1 * a1, a2 * a2)
    b0, b1, b2 = xt[0:1, :], xt[1:2, :], xt[2:3, :]
    x2c = _x2_sum(b0 * b0, b1 * b1, b2 * b2)

    if _DOT_MODE == "mxu":
        dot = lax.dot_general(
            xr, xt, (((1,), (0,)), ((), ())),
            precision=lax.Precision.DEFAULT,
            preferred_element_type=jnp.float32)
    else:
        dot = (a0 * b0 + a1 * b1) + a2 * b2

    d2 = (x2r + x2c) - 2.0 * dot
    m2 = vr * vc
    work = jnp.where(m2 > 0, d2, jnp.float32(_BIG))

    iota_i = lax.broadcasted_iota(jnp.int32, (1, n), 1)
    inf = jnp.float32(jnp.inf)
    big_i = jnp.int32(2**30)
    for k in range(_NUM_NEIGHBORS):
        mn = jnp.min(work, axis=1, keepdims=True)
        cand = jnp.where(work == mn, iota_i, big_i)
        idx = jnp.min(cand, axis=1, keepdims=True)
        ei_ref[0, k, :] = idx[:, 0]
        mk_ref[0, k, :] = jnp.where(mn[:, 0] < _BIG, 1.0, 0.0).astype(jnp.float32)
        work = jnp.where(iota_i == idx, inf, work)


_KPAD = 128


def _topk_edges(xca, xca_t, valid):
    b, n, _ = xca.shape
    grid = (b, n // _RB)
    return pl.pallas_call(
        _topk_body,
        grid=grid,
        in_specs=[
            pl.BlockSpec((1, _RB, _KPAD), lambda i, j: (i, j, 0)),
            pl.BlockSpec((1, _KPAD, n), lambda i, j: (i, 0, 0)),
            pl.BlockSpec((1, _RB, 1), lambda i, j: (i, j, 0)),
            pl.BlockSpec((1, 1, n), lambda i, j: (i, 0, 0)),
        ],
        out_specs=[
            pl.BlockSpec((1, _NUM_NEIGHBORS, _RB), lambda i, j: (i, 0, j)),
            pl.BlockSpec((1, _NUM_NEIGHBORS, _RB), lambda i, j: (i, 0, j)),
        ],
        out_shape=[
            jax.ShapeDtypeStruct((b, _NUM_NEIGHBORS, n), jnp.int32),
            jax.ShapeDtypeStruct((b, _NUM_NEIGHBORS, n), jnp.float32),
        ],
        compiler_params=pltpu.CompilerParams(
            dimension_semantics=("parallel", "parallel")),
    )(_pad_k(xca), jnp.transpose(_pad_k(xca), (0, 2, 1)),
      valid[:, :, None], valid[:, None, :])


def _pad_k(x):
    b, n, d = x.shape
    return jnp.concatenate(
        [x, jnp.zeros((b, n, _KPAD - d), x.dtype)], axis=-1)


def _smooth_sc(z0, edge_t, mask_t):
    b, n = z0.shape
    rps = _ROWS_PER_SUBCORE
    k = _NUM_NEIGHBORS
    mesh = plsc.VectorSubcoreMesh(core_axis_name="c", subcore_axis_name="s")

    @functools.partial(
        pl.kernel,
        out_type=jax.ShapeDtypeStruct((b, n), jnp.float32),
        mesh=mesh,
        scratch_types=[
            pltpu.VMEM((n,), jnp.float32),
            pltpu.VMEM((k, rps), jnp.int32),
            pltpu.VMEM((k, rps), jnp.float32),
            pltpu.VMEM((rps,), jnp.float32),
            pltpu.VMEM_SHARED((n,), jnp.float32),
        ],
        compiler_params=_sc_compiler_params(),
    )
    def smooth(z0_hbm, edge_hbm, mask_hbm, zout_hbm,
               z_loc, edge_loc, mask_loc, z_new, z_shared):
        c = lax.axis_index("c")
        s = lax.axis_index("s")
        base = s * rps
        pltpu.sync_copy(z0_hbm.at[c], z_loc)
        pltpu.sync_copy(edge_hbm.at[c, :, pl.ds(base, rps)], edge_loc)
        pltpu.sync_copy(mask_hbm.at[c, :, pl.ds(base, rps)], mask_loc)

        for _step in range(_SMOOTH_STEPS):
            @pl.loop(0, rps // _LANES)
            def _(g):
                off = g * _LANES
                terms = []
                msums = []
                for kk in range(k):
                    idx = edge_loc[kk, pl.ds(off, _LANES)]
                    mk = mask_loc[kk, pl.ds(off, _LANES)]
                    zg = plsc.load_gather(z_loc, [idx])
                    terms.append(zg * mk)
                    msums.append(mk)
                ssum = _tree_sum_k(terms)
                msum = _tree_sum_k(msums)
                z_new[pl.ds(off, _LANES)] = ssum / (msum + jnp.float32(_NORM_EPS))

            pltpu.sync_copy(z_new, z_shared.at[pl.ds(base, rps)])
            plsc.subcore_barrier()
            pltpu.sync_copy(z_shared, z_loc)
            plsc.subcore_barrier()

        @pl.when(s == 0)
        def _():
            pltpu.sync_copy(z_loc, zout_hbm.at[c])

    return smooth(z0, edge_t, mask_t)


def _rank_body(zr_ref, zc_ref, out_ref):
    j = pl.program_id(1)
    zr = zr_ref[0]
    zc = zc_ref[0]
    n = zc.shape[1]
    iota_j = lax.broadcasted_iota(jnp.int32, (1, n), 1)
    row_idx = j * _RB + lax.broadcasted_iota(jnp.int32, (_RB, 1), 0)
    lt = zc < zr
    eq_before = (zc == zr) & (iota_j < row_idx)
    cnt = jnp.sum(jnp.where(lt | eq_before, 1, 0).astype(jnp.int32),
                  axis=1)
    out_ref[0, 0, 0, :] = cnt


def _ranks(z):
    b, n = z.shape
    nblk = n // _RB
    out = pl.pallas_call(
        _rank_body,
        grid=(b, nblk),
        in_specs=[
            pl.BlockSpec((1, _RB, 1), lambda i, j: (i, j, 0)),
            pl.BlockSpec((1, 1, n), lambda i, j: (i, 0, 0)),
        ],
        out_specs=pl.BlockSpec((1, 1, 1, _RB), lambda i, j: (i, j, 0, 0)),
        out_shape=jax.ShapeDtypeStruct((b, nblk, 1, _RB), jnp.int32),
        compiler_params=pltpu.CompilerParams(
            dimension_semantics=("parallel", "parallel")),
    )(z[:, :, None], z[:, None, :])
    return out.reshape(b, n)


def _invert_perm_sc(rank):
    b, n = rank.shape
    mesh = plsc.VectorSubcoreMesh(core_axis_name="c", subcore_axis_name="s")

    @functools.partial(
        pl.kernel,
        out_type=jax.ShapeDtypeStruct((b, n), jnp.int32),
        mesh=mesh,
        scratch_types=[
            pltpu.VMEM((n,), jnp.int32),
            pltpu.VMEM((n,), jnp.int32),
        ],
        compiler_params=_sc_compiler_params(),
    )
    def invert(rank_hbm, out_hbm, idx_loc, out_loc):
        c = lax.axis_index("c")
        s = lax.axis_index("s")

        @pl.when(s == 0)
        def _():
            pltpu.sync_copy(rank_hbm.at[c], idx_loc)

            @pl.loop(0, n // _LANES)
            def _(i):
                off = i * _LANES
                idx = idx_loc[pl.ds(off, _LANES)]
                vals = lax.iota(jnp.int32, _LANES) + off
                plsc.store_scatter(out_loc, [idx], vals)

            pltpu.sync_copy(out_loc, out_hbm.at[c])

    return invert(rank)


def kernel(X, C):
    b, n = C.shape
    xca = X[:, :, 1, :]
    xca_t = jnp.transpose(xca, (0, 2, 1))
    valid = (C > 0).astype(jnp.float32)

    z = jax.random.uniform(jax.random.key(_DET_SEED), (1, n),
                           dtype=jnp.float32)
    z = jnp.broadcast_to(z, (b, n))

    edge_t, mask_t = _topk_edges(xca, xca_t, valid)
    z = _smooth_sc(z, edge_t, mask_t)
    rank = _ranks(z)
    return _invert_perm_sc(rank)


# speedup vs baseline: 18.0217x; 1.0704x over previous
import dataclasses
import functools

import jax
import jax.numpy as jnp
from jax import lax
from jax.experimental import pallas as pl
from jax.experimental.pallas import tpu as pltpu
from jax.experimental.pallas import tpu_sc as plsc

_NUM_NEIGHBORS = 30
_SMOOTH_STEPS = 5
_NORM_EPS = 1e-5
_DET_SEED = 10
_BIG = 1e9

_RB = 1024
_ROWS_PER_SUBCORE = 256
_LANES = 16

_DOT_MODE = "mxu"
_X2_MODE = "fold"
_K_MODE = "xla"


def _sc_compiler_params():
    cp = pltpu.CompilerParams()
    if "needs_layout_passes" in pltpu.CompilerParams.__dataclass_fields__:
        cp = dataclasses.replace(cp, needs_layout_passes=False)
    return cp


def _x2_sum(p0, p1, p2):
    if _X2_MODE == "fold":
        return (p0 + p2) + p1
    return (p0 + p1) + p2


def _tree_sum_k(ts):
    n = len(ts)
    if _K_MODE == "xla":
        u = []
        for s in range(8):
            acc = ts[s]
            j = s + 8
            while j < n:
                acc = acc + ts[j]
                j += 8
            u.append(acc)
        w = [u[s] + u[s + 4] for s in range(4)]
        return (w[0] + w[2]) + (w[1] + w[3])
    if _K_MODE == "seq":
        acc = ts[0]
        for t in ts[1:]:
            acc = acc + t
        return acc
    if _K_MODE == "fold":
        cur = list(ts) + [None] * (32 - n)
        width = 32
        while width > 1:
            half = width // 2
            nxt = []
            for i in range(half):
                a, b = cur[i], cur[i + half]
                if a is None:
                    nxt.append(b)
                elif b is None:
                    nxt.append(a)
                else:
                    nxt.append(a + b)
            cur = nxt
            width = half
        return cur[0]
    cur = list(ts)
    while len(cur) > 1:
        nxt = []
        for i in range(0, len(cur) - 1, 2):
            nxt.append(cur[i] + cur[i + 1])
        if len(cur) % 2:
            nxt.append(cur[-1])
        cur = nxt
    return cur[0]


def _topk_body(xr_ref, xt_ref, vr_ref, vc_ref, ei_ref, mk_ref):
    xr = xr_ref[0]
    xt = xt_ref[0]
    vr = vr_ref[0]
    vc = vc_ref[0]
    n = xt.shape[1]

    a0, a1, a2 = xr[:, 0:1], xr[:, 1:2], xr[:, 2:3]
    x2r = _x2_sum(a0 * a0, a1 * a1, a2 * a2)
    b0, b1, b2 = xt[0:1, :], xt[1:2, :], xt[2:3, :]
    x2c = _x2_sum(b0 * b0, b1 * b1, b2 * b2)

    if _DOT_MODE == "mxu":
        dot = lax.dot_general(
            xr, xt, (((1,), (0,)), ((), ())),
            precision=lax.Precision.DEFAULT,
            preferred_element_type=jnp.float32)
    else:
        dot = (a0 * b0 + a1 * b1) + a2 * b2

    d2 = (x2r + x2c) - 2.0 * dot
    m2 = vr * vc
    work = jnp.where(m2 > 0, d2, jnp.float32(_BIG))

    iota_i = lax.broadcasted_iota(jnp.int32, (1, n), 1)
    inf = jnp.float32(jnp.inf)
    big_i = jnp.int32(2**30)
    for k in range(_NUM_NEIGHBORS):
        mn = jnp.min(work, axis=1, keepdims=True)
        cand = jnp.where(work == mn, iota_i, big_i)
        idx = jnp.min(cand, axis=1, keepdims=True)
        ei_ref[0, k, :] = idx[:, 0]
        mk_ref[0, k, :] = jnp.where(mn[:, 0] < _BIG, 1.0, 0.0).astype(jnp.float32)
        work = jnp.where(iota_i == idx, inf, work)


_KPAD = 128


def _topk_edges(xca, xca_t, valid):
    b, n, _ = xca.shape
    grid = (b, n // _RB)
    return pl.pallas_call(
        _topk_body,
        grid=grid,
        in_specs=[
            pl.BlockSpec((1, _RB, _KPAD), lambda i, j: (i, j, 0)),
            pl.BlockSpec((1, _KPAD, n), lambda i, j: (i, 0, 0)),
            pl.BlockSpec((1, _RB, 1), lambda i, j: (i, j, 0)),
            pl.BlockSpec((1, 1, n), lambda i, j: (i, 0, 0)),
        ],
        out_specs=[
            pl.BlockSpec((1, _NUM_NEIGHBORS, _RB), lambda i, j: (i, 0, j)),
            pl.BlockSpec((1, _NUM_NEIGHBORS, _RB), lambda i, j: (i, 0, j)),
        ],
        out_shape=[
            jax.ShapeDtypeStruct((b, _NUM_NEIGHBORS, n), jnp.int32),
            jax.ShapeDtypeStruct((b, _NUM_NEIGHBORS, n), jnp.float32),
        ],
        compiler_params=pltpu.CompilerParams(
            dimension_semantics=("parallel", "parallel")),
    )(_pad_k(xca), jnp.transpose(_pad_k(xca), (0, 2, 1)),
      valid[:, :, None], valid[:, None, :])


def _pad_k(x):
    b, n, d = x.shape
    return jnp.concatenate(
        [x, jnp.zeros((b, n, _KPAD - d), x.dtype)], axis=-1)


def _smooth_sc(z0, edge_t, mask_t):
    b, n = z0.shape
    rps = _ROWS_PER_SUBCORE
    k = _NUM_NEIGHBORS
    mesh = plsc.VectorSubcoreMesh(core_axis_name="c", subcore_axis_name="s")

    @functools.partial(
        pl.kernel,
        out_type=jax.ShapeDtypeStruct((b, n), jnp.float32),
        mesh=mesh,
        scratch_types=[
            pltpu.VMEM((n,), jnp.float32),
            pltpu.VMEM((k, rps), jnp.int32),
            pltpu.VMEM((k, rps), jnp.float32),
            pltpu.VMEM((rps,), jnp.float32),
            pltpu.VMEM_SHARED((n,), jnp.float32),
        ],
        compiler_params=_sc_compiler_params(),
    )
    def smooth(z0_hbm, edge_hbm, mask_hbm, zout_hbm,
               z_loc, edge_loc, mask_loc, z_new, z_shared):
        c = lax.axis_index("c")
        s = lax.axis_index("s")
        base = s * rps
        pltpu.sync_copy(z0_hbm.at[c], z_loc)
        pltpu.sync_copy(edge_hbm.at[c, :, pl.ds(base, rps)], edge_loc)
        pltpu.sync_copy(mask_hbm.at[c, :, pl.ds(base, rps)], mask_loc)

        for _step in range(_SMOOTH_STEPS):
            @pl.loop(0, rps // _LANES)
            def _(g):
                off = g * _LANES
                terms = []
                msums = []
                for kk in range(k):
                    idx = edge_loc[kk, pl.ds(off, _LANES)]
                    mk = mask_loc[kk, pl.ds(off, _LANES)]
                    zg = plsc.load_gather(z_loc, [idx])
                    terms.append(zg * mk)
                    msums.append(mk)
                ssum = _tree_sum_k(terms)
                msum = _tree_sum_k(msums)
                z_new[pl.ds(off, _LANES)] = ssum / (msum + jnp.float32(_NORM_EPS))

            pltpu.sync_copy(z_new, z_shared.at[pl.ds(base, rps)])
            plsc.subcore_barrier()
            pltpu.sync_copy(z_shared, z_loc)
            plsc.subcore_barrier()

        @pl.when(s == 0)
        def _():
            pltpu.sync_copy(z_loc, zout_hbm.at[c])

    return smooth(z0, edge_t, mask_t)


def _rank_body(zr_ref, zc_ref, out_ref):
    j = pl.program_id(1)
    zr = zr_ref[0]
    zc = zc_ref[0]
    n = zc.shape[1]
    iota_j = lax.broadcasted_iota(jnp.int32, (1, n), 1)
    row_idx = j * _RB + lax.broadcasted_iota(jnp.int32, (_RB, 1), 0)
    lt = zc < zr
    eq_before = (zc == zr) & (iota_j < row_idx)
    cnt = jnp.sum(jnp.where(lt | eq_before, 1, 0).astype(jnp.int32),
                  axis=1)
    out_ref[0, 0, 0, :] = cnt


def _ranks(z):
    b, n = z.shape
    nblk = n // _RB
    out = pl.pallas_call(
        _rank_body,
        grid=(b, nblk),
        in_specs=[
            pl.BlockSpec((1, _RB, 1), lambda i, j: (i, j, 0)),
            pl.BlockSpec((1, 1, n), lambda i, j: (i, 0, 0)),
        ],
        out_specs=pl.BlockSpec((1, 1, 1, _RB), lambda i, j: (i, j, 0, 0)),
        out_shape=jax.ShapeDtypeStruct((b, nblk, 1, _RB), jnp.int32),
        compiler_params=pltpu.CompilerParams(
            dimension_semantics=("parallel", "parallel")),
    )(z[:, :, None], z[:, None, :])
    return out.reshape(b, n)


def _invert_perm_sc(rank):
    b, n = rank.shape
    mesh = plsc.VectorSubcoreMesh(core_axis_name="c", subcore_axis_name="s")

    @functools.partial(
        pl.kernel,
        out_type=jax.ShapeDtypeStruct((b, n), jnp.int32),
        mesh=mesh,
        scratch_types=[
            pltpu.VMEM((n,), jnp.int32),
            pltpu.VMEM((n,), jnp.int32),
        ],
        compiler_params=_sc_compiler_params(),
    )
    def invert(rank_hbm, out_hbm, idx_loc, out_loc):
        c = lax.axis_index("c")
        s = lax.axis_index("s")

        @pl.when(s == 0)
        def _():
            pltpu.sync_copy(rank_hbm.at[c], idx_loc)

            @pl.loop(0, n // _LANES)
            def _(i):
                off = i * _LANES
                idx = idx_loc[pl.ds(off, _LANES)]
                vals = lax.iota(jnp.int32, _LANES) + off
                plsc.store_scatter(out_loc, [idx], vals)

            pltpu.sync_copy(out_loc, out_hbm.at[c])

    return invert(rank)


def kernel(X, C):
    b, n = C.shape
    xca = X[:, :, 1, :]
    xca_t = jnp.transpose(xca, (0, 2, 1))
    valid = (C > 0).astype(jnp.float32)

    z = jax.random.uniform(jax.random.key(_DET_SEED), (1, n),
                           dtype=jnp.float32)
    z = jnp.broadcast_to(z, (b, n))

    edge_t, mask_t = _topk_edges(xca, xca_t, valid)
    z = _smooth_sc(z, edge_t, mask_t)
    rank = _ranks(z)
    return _invert_perm_sc(rank)
